# baseline (speedup 1.0000x reference)
import functools

import jax
import jax.numpy as jnp
from jax import lax
from jax.experimental import pallas as pl
from jax.experimental.pallas import tpu as pltpu
from jax.experimental.pallas import tpu_sc as plsc

D = 64
B = 16384
NW = 32
RPW = B // NW
GRP = RPW // 16
ROWW = 24 * D
BUFW = 16 * ROWW

LNCAT = 0
APAY = 896
WPAY = APAY + 1536
WNUM = WPAY + 64
BLN = WNUM + 64
ANUM = BLN + 64
C0PAY = ANUM + 896
C1PAY = C0PAY + 32
C0NSPL = C1PAY + 32
C1NSPL = C0NSPL + 224
C2PSPL = C1NSPL + 224
C2NSPL = C2PSPL + 16
NCONST = C2NSPL + 16
OFFC = (1, 3, 10)


def _rsqrt16(x):
    i = plsc.bitcast(x, jnp.int32)
    i = jnp.int32(0x5F3759DF) - (i >> 1)
    y = plsc.bitcast(i, jnp.float32)
    xh = x * 0.5
    for _ in range(3):
        y = y * (1.5 - xh * y * y)
    return y


def _sc_body(s_hbm, ic_hbm, ip_hbm, c_hbm, out_hbm,
             sv, icv, ipv, cv, ob, rqall, sems):
    wid = lax.axis_index("s") * 2 + lax.axis_index("c")
    base = wid * RPW
    pltpu.sync_copy(c_hbm, cv)
    pltpu.sync_copy(s_hbm.at[:, pl.ds(base, RPW)], sv)
    pltpu.sync_copy(ic_hbm.at[:, pl.ds(base, RPW)], icv)
    pltpu.sync_copy(ip_hbm.at[:, pl.ds(base, RPW)], ipv)

    for buf in range(2):
        for row in range(16):
            for k in range(4):
                ob[pl.ds(buf * BUFW + row * ROWW + k * 16, 16)] = \
                    cv[pl.ds(LNCAT + k * 16, 16)]

    iota = lax.iota(jnp.int32, 16)
    zeros = jnp.zeros((16,), jnp.int32)

    c2p = cv[pl.ds(C2PSPL, 16)]
    c2n = cv[pl.ds(C2NSPL, 16)]

    def phase1(gi, _):
        rbase = gi * 16
        goff = gi * 640
        for t in range(6):
            s = sv[t, pl.ds(rbase, 16)]
            iv = ipv[t, pl.ds(rbase, 16)]
            ci = iv + t * 4
            c0 = plsc.load_gather(cv, [ci + C0PAY])
            c1 = plsc.load_gather(cv, [ci + C1PAY])
            r = _rsqrt16((c2p * s + c1) * s + c0)
            rqall[pl.ds(goff + t * 16, 16)] = r
            rqall[pl.ds(goff + 320 + t * 16, 16)] = s * r
        for t in range(14):
            s = sv[6 + t, pl.ds(rbase, 16)]
            c0 = cv[pl.ds(C0NSPL + t * 16, 16)]
            c1 = cv[pl.ds(C1NSPL + t * 16, 16)]
            r = _rsqrt16((c2n * s + c1) * s + c0)
            rqall[pl.ds(goff + (6 + t) * 16, 16)] = r
            rqall[pl.ds(goff + 320 + (6 + t) * 16, 16)] = s * r
        return 0

    lax.fori_loop(0, GRP, phase1, 0)

    wp = [cv[pl.ds(WPAY + 16 * k, 16)] for k in range(4)]
    bl = [cv[pl.ds(BLN + 16 * k, 16)] for k in range(4)]
    wn = [cv[pl.ds(WNUM + 16 * k, 16)] for k in range(4)]

    def group(gi, _):
        buf = gi % 2
        boff = buf * BUFW

        @pl.when(gi >= 2)
        def _wait_prev():
            pltpu.make_async_copy(
                ob.at[pl.ds(boff, BUFW)],
                out_hbm.at[pl.ds(0, BUFW)],
                sems.at[buf]).wait()

        rbase = gi * 16
        goff = gi * 640

        for t in range(3):
            tspl = zeros + t
            for j in range(16):
                rowspl = (zeros + j) + rbase
                iv = plsc.load_gather(icv, [tspl, rowspl])
                gbj = iv * 64 + (LNCAT + OFFC[t] * 64)
                rowoff = boff + j * ROWW + (1 + t) * 64
                for k in range(4):
                    v = plsc.load_gather(cv, [gbj + (iota + 16 * k)])
                    ob[pl.ds(rowoff + 16 * k, 16)] = v

        for t in range(6):
            tspl = zeros + t
            for j in range(16):
                rowspl = (zeros + j) + rbase
                spl = (zeros + (t * 16 + j)) + goff
                rj = plsc.load_gather(rqall, [spl])
                qj = plsc.load_gather(rqall, [spl + 320])
                iv = plsc.load_gather(ipv, [tspl, rowspl])
                abj = iv * 64 + (APAY + t * 256)
                rowoff = boff + j * ROWW + (4 + t) * 64
                for k in range(4):
                    a = plsc.load_gather(cv, [abj + (iota + 16 * k)])
                    ob[pl.ds(rowoff + 16 * k, 16)] = a * rj + wp[k] * qj + bl[k]

        for t in range(14):
            ak = [cv[pl.ds(ANUM + t * 64 + 16 * k, 16)] for k in range(4)]
            for j in range(16):
                spl = (zeros + ((6 + t) * 16 + j)) + goff
                rj = plsc.load_gather(rqall, [spl])
                qj = plsc.load_gather(rqall, [spl + 320])
                rowoff = boff + j * ROWW + (10 + t) * 64
                for k in range(4):
                    ob[pl.ds(rowoff + 16 * k, 16)] = ak[k] * rj + wn[k] * qj + bl[k]

        pltpu.async_copy(
            ob.at[pl.ds(boff, BUFW)],
            out_hbm.at[pl.ds((base + rbase) * ROWW, BUFW)],
            sems.at[buf])
        return 0

    lax.fori_loop(0, GRP, group, 0)
    pltpu.make_async_copy(ob.at[pl.ds(0, BUFW)],
                          out_hbm.at[pl.ds(0, BUFW)], sems.at[0]).wait()
    pltpu.make_async_copy(ob.at[pl.ds(BUFW, BUFW)],
                          out_hbm.at[pl.ds(0, BUFW)], sems.at[1]).wait()


@functools.partial(jax.jit, static_argnums=())
def _run_sc(s_all, ic, ip, consts):
    mesh = plsc.VectorSubcoreMesh(core_axis_name="c", subcore_axis_name="s",
                                  num_cores=2, num_subcores=16)
    k = pl.kernel(
        _sc_body,
        out_type=jax.ShapeDtypeStruct((B * ROWW,), jnp.float32),
        mesh=mesh,
        compiler_params=pltpu.CompilerParams(needs_layout_passes=False),
        scratch_types=[
            pltpu.VMEM((20, RPW), jnp.float32),
            pltpu.VMEM((3, RPW), jnp.int32),
            pltpu.VMEM((6, RPW), jnp.int32),
            pltpu.VMEM((NCONST,), jnp.float32),
            pltpu.VMEM((2 * BUFW,), jnp.float32),
            pltpu.VMEM((GRP * 640,), jnp.float32),
            pltpu.SemaphoreType.DMA((2,)),
        ],
    )
    return k(s_all, ic, ip, consts)


def kernel(cat_idx_sex, cat_idx_education, cat_idx_marriage, pay_state_ids,
           pay_severities, num_values, emb_sex, emb_education, emb_marriage,
           pay_state_table, sev_W, sev_b, num_feat_table, val_W, val_b,
           pos_table, cls_token, ln_g, ln_b):
    f32 = jnp.float32
    g = ln_g.astype(f32)
    bln = ln_b.astype(f32)
    pos = pos_table.astype(f32)
    eps = 1e-5

    rows = jnp.concatenate([
        (cls_token[0, 0] + pos[0])[None],
        emb_sex + pos[1], emb_education + pos[2], emb_marriage + pos[3],
    ], axis=0)
    mu = rows.mean(-1, keepdims=True)
    var = ((rows - mu) ** 2).mean(-1, keepdims=True)
    lncat = (rows - mu) * lax.rsqrt(var + eps) * g + bln

    w_pay = sev_W[:, 0]
    a_pay = pay_state_table[None, :, :] + sev_b + pos[4:10][:, None, :]
    ah_pay = a_pay - a_pay.mean(-1, keepdims=True)
    wh_pay = w_pay - w_pay.mean()
    c0_pay = (ah_pay ** 2).mean(-1) + eps
    c1_pay = 2.0 * (ah_pay * wh_pay).mean(-1)
    c2_pay = (wh_pay ** 2).mean()

    w_num = val_W[:, 0]
    a_num = num_feat_table + val_b + pos[10:24]
    ah_num = a_num - a_num.mean(-1, keepdims=True)
    wh_num = w_num - w_num.mean()
    c0_num = (ah_num ** 2).mean(-1) + eps
    c1_num = 2.0 * (ah_num * wh_num).mean(-1)
    c2_num = (wh_num ** 2).mean()

    pad8 = jnp.zeros((8,), f32)
    consts = jnp.concatenate([
        lncat.reshape(-1),
        (ah_pay * g).reshape(-1),
        wh_pay * g, wh_num * g, bln,
        (ah_num * g).reshape(-1),
        c0_pay.reshape(-1), pad8, c1_pay.reshape(-1), pad8,
        jnp.repeat(c0_num, 16), jnp.repeat(c1_num, 16),
        jnp.full((16,), c2_pay, f32), jnp.full((16,), c2_num, f32),
    ])

    s_all = jnp.concatenate([pay_severities.T, num_values.T], axis=0)
    ic = jnp.stack([cat_idx_sex, cat_idx_education,
                    cat_idx_marriage]).astype(jnp.int32)
    ip = pay_state_ids.T.astype(jnp.int32)

    out = _run_sc(s_all.astype(f32), ic, ip, consts)
    return out.reshape(B, 24, D)


# speedup vs baseline: 1.2917x; 1.2917x over previous
import functools

import jax
import jax.numpy as jnp
from jax import lax
from jax.experimental import pallas as pl
from jax.experimental.pallas import tpu as pltpu
from jax.experimental.pallas import tpu_sc as plsc

D = 64
B = 16384
NW = 32
RPW = B // NW
GRP = RPW // 16
ROWW = 24 * D
BUFW = 16 * ROWW

LNCAT = 0
APAY = 896
WPAY = APAY + 1536
WNUM = WPAY + 64
BLN = WNUM + 64
ANUM = BLN + 64
C0PAY = ANUM + 896
C1PAY = C0PAY + 32
C0NSPL = C1PAY + 32
C1NSPL = C0NSPL + 224
C2PSPL = C1NSPL + 224
C2NSPL = C2PSPL + 16
NCONST = C2NSPL + 16
OFFC = (1, 3, 10)


def _rsqrt16(x):
    i = plsc.bitcast(x, jnp.int32)
    i = jnp.int32(0x5F3759DF) - (i >> 1)
    y = plsc.bitcast(i, jnp.float32)
    xh = x * 0.5
    for _ in range(3):
        y = y * (1.5 - xh * y * y)
    return y


def _sc_body(s_hbm, ic_hbm, ip_hbm, c_hbm, out_hbm,
             sv, icv, ipv, cv, ob, rqall, sems):
    wid = lax.axis_index("s") * 2 + lax.axis_index("c")
    base = wid * RPW
    pltpu.sync_copy(c_hbm, cv)
    pltpu.sync_copy(s_hbm.at[:, pl.ds(base, RPW)], sv)
    pltpu.sync_copy(ic_hbm.at[:, pl.ds(base, RPW)], icv)
    pltpu.sync_copy(ip_hbm.at[:, pl.ds(base, RPW)], ipv)

    for row in range(32):
        for k in range(4):
            ob[row, pl.ds(k * 16, 16)] = cv[pl.ds(LNCAT + k * 16, 16)]

    iota = lax.iota(jnp.int32, 16)
    zeros = jnp.zeros((16,), jnp.int32)

    c2p = cv[pl.ds(C2PSPL, 16)]
    c2n = cv[pl.ds(C2NSPL, 16)]

    def phase1(gi, _):
        rbase = gi * 16
        goff = gi * 640
        for t in range(6):
            s = sv[t, pl.ds(rbase, 16)]
            iv = ipv[t, pl.ds(rbase, 16)]
            ci = iv + t * 4
            c0 = plsc.load_gather(cv, [ci + C0PAY])
            c1 = plsc.load_gather(cv, [ci + C1PAY])
            r = _rsqrt16((c2p * s + c1) * s + c0)
            rqall[pl.ds(goff + t * 16, 16)] = r
            rqall[pl.ds(goff + 320 + t * 16, 16)] = s * r
        for t in range(14):
            s = sv[6 + t, pl.ds(rbase, 16)]
            c0 = cv[pl.ds(C0NSPL + t * 16, 16)]
            c1 = cv[pl.ds(C1NSPL + t * 16, 16)]
            r = _rsqrt16((c2n * s + c1) * s + c0)
            rqall[pl.ds(goff + (6 + t) * 16, 16)] = r
            rqall[pl.ds(goff + 320 + (6 + t) * 16, 16)] = s * r
        return 0

    lax.fori_loop(0, GRP, phase1, 0)

    wp = [cv[pl.ds(WPAY + 16 * k, 16)] for k in range(4)]
    bl = [cv[pl.ds(BLN + 16 * k, 16)] for k in range(4)]
    wn = [cv[pl.ds(WNUM + 16 * k, 16)] for k in range(4)]

    def group(gi, _):
        buf = gi % 2
        brow0 = buf * 16

        @pl.when(gi >= 2)
        def _wait_prev():
            pltpu.make_async_copy(
                ob.at[pl.ds(brow0, 16), :],
                out_hbm.at[pl.ds(0, 16), :],
                sems.at[buf]).wait()

        rbase = gi * 16
        goff = gi * 640

        for t in range(3):
            tspl = zeros + t
            for j in range(16):
                rowspl = (zeros + j) + rbase
                iv = plsc.load_gather(icv, [tspl, rowspl])
                gbj = iv * 64 + (LNCAT + OFFC[t] * 64)
                brow = brow0 + j
                for k in range(4):
                    v = plsc.load_gather(cv, [gbj + (iota + 16 * k)])
                    ob[brow, pl.ds((1 + t) * 64 + 16 * k, 16)] = v

        for t in range(6):
            tspl = zeros + t
            for j in range(16):
                rowspl = (zeros + j) + rbase
                spl = (zeros + (t * 16 + j)) + goff
                rj = plsc.load_gather(rqall, [spl])
                qj = plsc.load_gather(rqall, [spl + 320])
                iv = plsc.load_gather(ipv, [tspl, rowspl])
                abj = iv * 64 + (APAY + t * 256)
                brow = brow0 + j
                for k in range(4):
                    a = plsc.load_gather(cv, [abj + (iota + 16 * k)])
                    ob[brow, pl.ds((4 + t) * 64 + 16 * k, 16)] = \
                        a * rj + wp[k] * qj + bl[k]

        for t in range(14):
            ak = [cv[pl.ds(ANUM + t * 64 + 16 * k, 16)] for k in range(4)]
            for j in range(16):
                spl = (zeros + ((6 + t) * 16 + j)) + goff
                rj = plsc.load_gather(rqall, [spl])
                qj = plsc.load_gather(rqall, [spl + 320])
                brow = brow0 + j
                for k in range(4):
                    ob[brow, pl.ds((10 + t) * 64 + 16 * k, 16)] = \
                        ak[k] * rj + wn[k] * qj + bl[k]

        pltpu.async_copy(
            ob.at[pl.ds(brow0, 16), :],
            out_hbm.at[pl.ds(base + rbase, 16), :],
            sems.at[buf])
        return 0

    lax.fori_loop(0, GRP, group, 0)
    pltpu.make_async_copy(ob.at[pl.ds(0, 16), :],
                          out_hbm.at[pl.ds(0, 16), :], sems.at[0]).wait()
    pltpu.make_async_copy(ob.at[pl.ds(16, 16), :],
                          out_hbm.at[pl.ds(0, 16), :], sems.at[1]).wait()


@functools.partial(jax.jit, static_argnums=())
def _run_sc(s_all, ic, ip, consts):
    mesh = plsc.VectorSubcoreMesh(core_axis_name="c", subcore_axis_name="s",
                                  num_cores=2, num_subcores=16)
    k = pl.kernel(
        _sc_body,
        out_type=jax.ShapeDtypeStruct((B, ROWW), jnp.float32),
        mesh=mesh,
        compiler_params=pltpu.CompilerParams(needs_layout_passes=False),
        scratch_types=[
            pltpu.VMEM((20, RPW), jnp.float32),
            pltpu.VMEM((3, RPW), jnp.int32),
            pltpu.VMEM((6, RPW), jnp.int32),
            pltpu.VMEM((NCONST,), jnp.float32),
            pltpu.VMEM((32, ROWW), jnp.float32),
            pltpu.VMEM((GRP * 640,), jnp.float32),
            pltpu.SemaphoreType.DMA((2,)),
        ],
    )
    return k(s_all, ic, ip, consts)


def kernel(cat_idx_sex, cat_idx_education, cat_idx_marriage, pay_state_ids,
           pay_severities, num_values, emb_sex, emb_education, emb_marriage,
           pay_state_table, sev_W, sev_b, num_feat_table, val_W, val_b,
           pos_table, cls_token, ln_g, ln_b):
    f32 = jnp.float32
    g = ln_g.astype(f32)
    bln = ln_b.astype(f32)
    pos = pos_table.astype(f32)
    eps = 1e-5

    rows = jnp.concatenate([
        (cls_token[0, 0] + pos[0])[None],
        emb_sex + pos[1], emb_education + pos[2], emb_marriage + pos[3],
    ], axis=0)
    mu = rows.mean(-1, keepdims=True)
    var = ((rows - mu) ** 2).mean(-1, keepdims=True)
    lncat = (rows - mu) * lax.rsqrt(var + eps) * g + bln

    w_pay = sev_W[:, 0]
    a_pay = pay_state_table[None, :, :] + sev_b + pos[4:10][:, None, :]
    ah_pay = a_pay - a_pay.mean(-1, keepdims=True)
    wh_pay = w_pay - w_pay.mean()
    c0_pay = (ah_pay ** 2).mean(-1) + eps
    c1_pay = 2.0 * (ah_pay * wh_pay).mean(-1)
    c2_pay = (wh_pay ** 2).mean()

    w_num = val_W[:, 0]
    a_num = num_feat_table + val_b + pos[10:24]
    ah_num = a_num - a_num.mean(-1, keepdims=True)
    wh_num = w_num - w_num.mean()
    c0_num = (ah_num ** 2).mean(-1) + eps
    c1_num = 2.0 * (ah_num * wh_num).mean(-1)
    c2_num = (wh_num ** 2).mean()

    pad8 = jnp.zeros((8,), f32)
    consts = jnp.concatenate([
        lncat.reshape(-1),
        (ah_pay * g).reshape(-1),
        wh_pay * g, wh_num * g, bln,
        (ah_num * g).reshape(-1),
        c0_pay.reshape(-1), pad8, c1_pay.reshape(-1), pad8,
        jnp.repeat(c0_num, 16), jnp.repeat(c1_num, 16),
        jnp.full((16,), c2_pay, f32), jnp.full((16,), c2_num, f32),
    ])

    s_all = jnp.concatenate([pay_severities.T, num_values.T], axis=0)
    ic = jnp.stack([cat_idx_sex, cat_idx_education,
                    cat_idx_marriage]).astype(jnp.int32)
    ip = pay_state_ids.T.astype(jnp.int32)

    out = _run_sc(s_all.astype(f32), ic, ip, consts)
    return out.reshape(B, 24, D)


# speedup vs baseline: 1.4562x; 1.1273x over previous
import functools

import jax
import jax.numpy as jnp
from jax import lax
from jax.experimental import pallas as pl
from jax.experimental.pallas import tpu as pltpu
from jax.experimental.pallas import tpu_sc as plsc

D = 64
B = 16384
NW = 32
RPW = B // NW
GRP = RPW // 16
ROWW = 24 * D
BUFW = 16 * ROWW

LNCAT = 0
APAY = 896
WPAY = APAY + 1536
WNUM = WPAY + 64
BLN = WNUM + 64
ANUM = BLN + 64
C0PAY = ANUM + 896
C1PAY = C0PAY + 32
C0NSPL = C1PAY + 32
C1NSPL = C0NSPL + 224
C2PSPL = C1NSPL + 224
C2NSPL = C2PSPL + 16
NCONST = C2NSPL + 16
OFFC = (1, 3, 10)
RQP = 10240
IRP = 2304


def _rsqrt16(x):
    i = plsc.bitcast(x, jnp.int32)
    i = jnp.int32(0x5F3759DF) - (i >> 1)
    y = plsc.bitcast(i, jnp.float32)
    xh = x * 0.5
    for _ in range(3):
        y = y * (1.5 - xh * y * y)
    return y


def _sc_body(s_hbm, ic_hbm, ip_hbm, c_hbm, out_hbm,
             sv, icv, ipv, cv, ob, rrep, irep, sems):
    wid = lax.axis_index("s") * 2 + lax.axis_index("c")
    base = wid * RPW
    pltpu.sync_copy(c_hbm, cv)
    pltpu.sync_copy(s_hbm.at[:, pl.ds(base, RPW)], sv)
    pltpu.sync_copy(ic_hbm.at[:, pl.ds(base, RPW)], icv)
    pltpu.sync_copy(ip_hbm.at[:, pl.ds(base, RPW)], ipv)

    for row in range(32):
        for k in range(4):
            ob[row, pl.ds(k * 16, 16)] = cv[pl.ds(LNCAT + k * 16, 16)]

    iota = lax.iota(jnp.int32, 16)

    def repl_f(vec, off):
        def body(c, _):
            dg = iota * 16 + ((iota + c) & 15)
            plsc.store_scatter(rrep, [dg + off], vec)
            return 0
        lax.fori_loop(0, 16, body, 0, unroll=4)

    def repl_i(vec, off):
        def body(c, _):
            dg = iota * 16 + ((iota + c) & 15)
            plsc.store_scatter(irep, [dg + off], vec)
            return 0
        lax.fori_loop(0, 16, body, 0, unroll=4)

    c2p = cv[pl.ds(C2PSPL, 16)]
    c2n = cv[pl.ds(C2NSPL, 16)]

    def phase1(gi):
        p = gi % 2
        rbase = gi * 16
        ro = p * RQP
        io = p * IRP
        for t in range(3):
            iv = icv[t, pl.ds(rbase, 16)]
            repl_i(iv * 64 + (LNCAT + OFFC[t] * 64), io + t * 256)
        for t in range(6):
            s = sv[t, pl.ds(rbase, 16)]
            iv = ipv[t, pl.ds(rbase, 16)]
            ci = iv + t * 4
            c0 = plsc.load_gather(cv, [ci + C0PAY])
            c1 = plsc.load_gather(cv, [ci + C1PAY])
            r = _rsqrt16((c2p * s + c1) * s + c0)
            repl_i(iv * 64 + (APAY + t * 256), io + (3 + t) * 256)
            repl_f(r, ro + t * 256)
            repl_f(s * r, ro + 5120 + t * 256)
        for t in range(14):
            s = sv[6 + t, pl.ds(rbase, 16)]
            c0 = cv[pl.ds(C0NSPL + t * 16, 16)]
            c1 = cv[pl.ds(C1NSPL + t * 16, 16)]
            r = _rsqrt16((c2n * s + c1) * s + c0)
            repl_f(r, ro + (6 + t) * 256)
            repl_f(s * r, ro + 5120 + (6 + t) * 256)

    wp = [cv[pl.ds(WPAY + 16 * k, 16)] for k in range(4)]
    bl = [cv[pl.ds(BLN + 16 * k, 16)] for k in range(4)]
    wn = [cv[pl.ds(WNUM + 16 * k, 16)] for k in range(4)]

    phase1(0)

    def group(gi, _):
        buf = gi % 2
        brow0 = buf * 16
        p = gi % 2
        ro = p * RQP
        io = p * IRP

        @pl.when(gi >= 2)
        def _wait_prev():
            pltpu.make_async_copy(
                ob.at[pl.ds(brow0, 16), :],
                out_hbm.at[pl.ds(0, 16), :],
                sems.at[buf]).wait()

        rbase = gi * 16

        for t in range(3):
            def catj(j, _, t=t):
                gbj = irep[pl.ds(io + t * 256 + j * 16, 16)]
                brow = brow0 + j
                for k in range(4):
                    v = plsc.load_gather(cv, [gbj + (iota + 16 * k)])
                    ob[brow, pl.ds((1 + t) * 64 + 16 * k, 16)] = v
                return 0
            lax.fori_loop(0, 16, catj, 0, unroll=4)

        for t in range(6):
            def payj(j, _, t=t):
                rj = rrep[pl.ds(ro + t * 256 + j * 16, 16)]
                qj = rrep[pl.ds(ro + 5120 + t * 256 + j * 16, 16)]
                abj = irep[pl.ds(io + (3 + t) * 256 + j * 16, 16)]
                brow = brow0 + j
                for k in range(4):
                    a = plsc.load_gather(cv, [abj + (iota + 16 * k)])
                    ob[brow, pl.ds((4 + t) * 64 + 16 * k, 16)] = \
                        a * rj + wp[k] * qj + bl[k]
                return 0
            lax.fori_loop(0, 16, payj, 0, unroll=4)

        for t in range(14):
            ak = [cv[pl.ds(ANUM + t * 64 + 16 * k, 16)] for k in range(4)]

            def numj(j, _, t=t, ak=ak):
                rj = rrep[pl.ds(ro + (6 + t) * 256 + j * 16, 16)]
                qj = rrep[pl.ds(ro + 5120 + (6 + t) * 256 + j * 16, 16)]
                brow = brow0 + j
                for k in range(4):
                    ob[brow, pl.ds((10 + t) * 64 + 16 * k, 16)] = \
                        ak[k] * rj + wn[k] * qj + bl[k]
                return 0
            lax.fori_loop(0, 16, numj, 0, unroll=4)

        pltpu.async_copy(
            ob.at[pl.ds(brow0, 16), :],
            out_hbm.at[pl.ds(base + rbase, 16), :],
            sems.at[buf])

        @pl.when(gi + 1 < GRP)
        def _next_phase1():
            phase1(gi + 1)

        return 0

    lax.fori_loop(0, GRP, group, 0)
    pltpu.make_async_copy(ob.at[pl.ds(0, 16), :],
                          out_hbm.at[pl.ds(0, 16), :], sems.at[0]).wait()
    pltpu.make_async_copy(ob.at[pl.ds(16, 16), :],
                          out_hbm.at[pl.ds(0, 16), :], sems.at[1]).wait()


@functools.partial(jax.jit, static_argnums=())
def _run_sc(s_all, ic, ip, consts):
    mesh = plsc.VectorSubcoreMesh(core_axis_name="c", subcore_axis_name="s",
                                  num_cores=2, num_subcores=16)
    k = pl.kernel(
        _sc_body,
        out_type=jax.ShapeDtypeStruct((B, ROWW), jnp.float32),
        mesh=mesh,
        compiler_params=pltpu.CompilerParams(needs_layout_passes=False),
        scratch_types=[
            pltpu.VMEM((20, RPW), jnp.float32),
            pltpu.VMEM((3, RPW), jnp.int32),
            pltpu.VMEM((6, RPW), jnp.int32),
            pltpu.VMEM((NCONST,), jnp.float32),
            pltpu.VMEM((32, ROWW), jnp.float32),
            pltpu.VMEM((2 * RQP,), jnp.float32),
            pltpu.VMEM((2 * IRP,), jnp.int32),
            pltpu.SemaphoreType.DMA((2,)),
        ],
    )
    return k(s_all, ic, ip, consts)


def kernel(cat_idx_sex, cat_idx_education, cat_idx_marriage, pay_state_ids,
           pay_severities, num_values, emb_sex, emb_education, emb_marriage,
           pay_state_table, sev_W, sev_b, num_feat_table, val_W, val_b,
           pos_table, cls_token, ln_g, ln_b):
    f32 = jnp.float32
    g = ln_g.astype(f32)
    bln = ln_b.astype(f32)
    pos = pos_table.astype(f32)
    eps = 1e-5

    rows = jnp.concatenate([
        (cls_token[0, 0] + pos[0])[None],
        emb_sex + pos[1], emb_education + pos[2], emb_marriage + pos[3],
    ], axis=0)
    mu = rows.mean(-1, keepdims=True)
    var = ((rows - mu) ** 2).mean(-1, keepdims=True)
    lncat = (rows - mu) * lax.rsqrt(var + eps) * g + bln

    w_pay = sev_W[:, 0]
    a_pay = pay_state_table[None, :, :] + sev_b + pos[4:10][:, None, :]
    ah_pay = a_pay - a_pay.mean(-1, keepdims=True)
    wh_pay = w_pay - w_pay.mean()
    c0_pay = (ah_pay ** 2).mean(-1) + eps
    c1_pay = 2.0 * (ah_pay * wh_pay).mean(-1)
    c2_pay = (wh_pay ** 2).mean()

    w_num = val_W[:, 0]
    a_num = num_feat_table + val_b + pos[10:24]
    ah_num = a_num - a_num.mean(-1, keepdims=True)
    wh_num = w_num - w_num.mean()
    c0_num = (ah_num ** 2).mean(-1) + eps
    c1_num = 2.0 * (ah_num * wh_num).mean(-1)
    c2_num = (wh_num ** 2).mean()

    pad8 = jnp.zeros((8,), f32)
    consts = jnp.concatenate([
        lncat.reshape(-1),
        (ah_pay * g).reshape(-1),
        wh_pay * g, wh_num * g, bln,
        (ah_num * g).reshape(-1),
        c0_pay.reshape(-1), pad8, c1_pay.reshape(-1), pad8,
        jnp.repeat(c0_num, 16), jnp.repeat(c1_num, 16),
        jnp.full((16,), c2_pay, f32), jnp.full((16,), c2_num, f32),
    ])

    s_all = jnp.concatenate([pay_severities.T, num_values.T], axis=0)
    ic = jnp.stack([cat_idx_sex, cat_idx_education,
                    cat_idx_marriage]).astype(jnp.int32)
    ip = pay_state_ids.T.astype(jnp.int32)

    out = _run_sc(s_all.astype(f32), ic, ip, consts)
    return out.reshape(B, 24, D)
